# baseline (speedup 1.0000x reference)
import functools

import jax
import jax.numpy as jnp
from jax import lax
from jax.experimental import pallas as pl
from jax.experimental.pallas import tpu as pltpu
from jax.experimental.pallas import tpu_sc as plsc

BATCH = 4096
HIST = 50
EMBED_DIM = 64

NUM_CORES = 2
NUM_SUBCORES = 16
NUM_WORKERS = NUM_CORES * NUM_SUBCORES

ROWS_PER_WORKER = BATCH // NUM_WORKERS
NCHUNK = ROWS_PER_WORKER
NBUF = 8


@jax.jit
def _sc_embedding_lookup(ids, table):
    mesh = plsc.VectorSubcoreMesh(
        core_axis_name="c", subcore_axis_name="s",
        num_cores=NUM_CORES, num_subcores=NUM_SUBCORES)

    @functools.partial(
        pl.kernel,
        out_type=jax.ShapeDtypeStruct((BATCH, HIST, EMBED_DIM), jnp.float32),
        mesh=mesh,
        scratch_types=[
            pltpu.VMEM((ROWS_PER_WORKER, HIST), jnp.int32),
            pltpu.VMEM((NBUF, HIST, EMBED_DIM), jnp.float32),
            pltpu.SemaphoreType.DMA((NBUF,)),
            pltpu.SemaphoreType.DMA((NBUF,)),
        ],
        compiler_params=pltpu.CompilerParams(use_tc_tiling_on_sc=False),
    )
    def k(ids_hbm, table_hbm, out_hbm, idx_v, rows_v, gsems, wsems):
        wid = lax.axis_index("s") * NUM_CORES + lax.axis_index("c")
        base = wid * ROWS_PER_WORKER
        pltpu.sync_copy(ids_hbm.at[pl.ds(base, ROWS_PER_WORKER)], idx_v)

        for b in range(NBUF):
            pltpu.async_copy(table_hbm.at[idx_v.at[b]], rows_v.at[b],
                             gsems.at[b])

        def outer(g, _):
            for b in range(NBUF):
                j = g * NBUF + b
                pltpu.make_async_copy(
                    table_hbm.at[idx_v.at[j]], rows_v.at[b],
                    gsems.at[b]).wait()
                pltpu.async_copy(rows_v.at[b], out_hbm.at[base + j],
                                 wsems.at[b])
            for b in range(NBUF):
                j = g * NBUF + b

                @pl.when(j + NBUF < NCHUNK)
                def _():
                    pltpu.make_async_copy(
                        rows_v.at[b], out_hbm.at[base + j],
                        wsems.at[b]).wait()
                    pltpu.async_copy(table_hbm.at[idx_v.at[j + NBUF]],
                                     rows_v.at[b], gsems.at[b])

            return 0

        lax.fori_loop(0, NCHUNK // NBUF, outer, 0)

        for b in range(NBUF):
            j = NCHUNK - NBUF + b
            pltpu.make_async_copy(rows_v.at[b], out_hbm.at[base + j],
                                  wsems.at[b]).wait()

    return k(ids, table)


def kernel(satellite_ids, embedding_table):
    return _sc_embedding_lookup(satellite_ids, embedding_table)
